# baseline (speedup 1.0000x reference)
import jax
import jax.numpy as jnp
from jax.experimental import pallas as pl
from jax.experimental.pallas import tpu as pltpu

_B = 4
_C = 384
_H = 224
_W = 224
_HW = _H * _W
_R = 48
_E = 64
_K = 8
_EPS = 1e-5

_HBLK = 14
_NH = _H // _HBLK
_NBUF = 4


def _route(pooled, w1_ref, b1_ref, w2_ref, b2_ref, vals_ref, idx_ref):
    hid = jax.lax.dot_general(pooled, w1_ref[...], (((1,), (1,)), ((), ())),
                              preferred_element_type=jnp.float32)
    hid = hid + b1_ref[...]
    hid = hid * jax.nn.sigmoid(hid)
    logits = jax.lax.dot_general(hid, w2_ref[...], (((1,), (1,)), ((), ())),
                                 preferred_element_type=jnp.float32)
    logits = logits + b2_ref[...]
    m = jnp.max(logits, axis=1, keepdims=True)
    e = jnp.exp(logits - m)
    probs = e / jnp.sum(e, axis=1, keepdims=True)

    pa = probs[:, :, None]
    pb = probs[:, None, :]
    ks = jax.lax.broadcasted_iota(jnp.int32, (_B, _E, _E), 1)
    js = jax.lax.broadcasted_iota(jnp.int32, (_B, _E, _E), 2)
    beats = (pa > pb) | ((pa == pb) & (ks < js))
    rank = jnp.sum(beats.astype(jnp.int32), axis=1)

    iota = jax.lax.broadcasted_iota(jnp.int32, (_B, _E), 1)
    vals = []
    idxs = []
    for s in range(_K):
        sel = rank == s
        vals.append(jnp.sum(jnp.where(sel, probs, 0.0), axis=1, keepdims=True))
        idxs.append(jnp.sum(jnp.where(sel, iota, 0), axis=1, keepdims=True))
    v = jnp.concatenate(vals, axis=1)
    i = jnp.concatenate(idxs, axis=1)
    ssum = jnp.sum(v, axis=1, keepdims=True) + 1e-6
    vals_ref[...] = v / ssum
    idx_ref[...] = i


def _body(xt_ref, w1_ref, b1_ref, w2_ref, b2_ref, vals_ref, idx_ref,
          r0, r1, r2, r3, r4, r5, r6, r7, sums_ref, sems):
    rings = (r0, r1, r2, r3, r4, r5, r6, r7)

    def start(b, h, parity):
        pltpu.make_async_copy(
            xt_ref.at[b, pl.ds(h * _HBLK, _HBLK)],
            rings[b * 2 + parity],
            sems.at[b * 2 + parity],
        ).start()

    def wait(b, parity):
        pltpu.make_async_copy(
            xt_ref.at[b, pl.ds(0, _HBLK)],
            rings[b * 2 + parity],
            sems.at[b * 2 + parity],
        ).wait()

    for b in range(4):
        start(b, 0, 0)
    for b in range(4):
        start(b, 1, 1)

    def step(g, _):
        for parity in range(2):
            h = 2 * g + parity
            for b in range(4):
                wait(b, parity)
                s = jnp.sum(rings[b * 2 + parity][...], axis=0)
                part = jnp.sum(s, axis=0)

                @pl.when(h == 0)
                def _init():
                    sums_ref[b, :] = part

                @pl.when(h != 0)
                def _acc():
                    sums_ref[b, :] += part

                @pl.when(h + 2 < _NH)
                def _prefetch():
                    start(b, h + 2, parity)
        return _

    jax.lax.fori_loop(0, _NH // 2, step, None)
    _route(sums_ref[...], w1_ref, b1_ref, w2_ref, b2_ref, vals_ref, idx_ref)


@jax.jit
def kernel(x, W1, gamma1, beta1, mean1, var1, W2, gamma2, beta2, mean2, var2):
    s1 = gamma1 * jax.lax.rsqrt(var1 + _EPS)
    s2 = gamma2 * jax.lax.rsqrt(var2 + _EPS)
    w1 = (W1 * s1[:, None]) * (1.0 / _HW)
    b1 = (beta1 - mean1 * s1)[None, :]
    w2 = W2 * s2[:, None]
    b2 = (beta2 - mean2 * s2)[None, :]

    xt = jnp.transpose(x, (0, 2, 3, 1))
    vals, idxs = pl.pallas_call(
        _body,
        in_specs=[
            pl.BlockSpec(memory_space=pl.ANY),
            pl.BlockSpec(memory_space=pltpu.VMEM),
            pl.BlockSpec(memory_space=pltpu.VMEM),
            pl.BlockSpec(memory_space=pltpu.VMEM),
            pl.BlockSpec(memory_space=pltpu.VMEM),
        ],
        out_specs=(
            pl.BlockSpec(memory_space=pltpu.VMEM),
            pl.BlockSpec(memory_space=pltpu.VMEM),
        ),
        out_shape=(
            jax.ShapeDtypeStruct((_B, _K), jnp.float32),
            jax.ShapeDtypeStruct((_B, _K), jnp.int32),
        ),
        scratch_shapes=[
            pltpu.VMEM((_HBLK, _W, _C), jnp.float32),
            pltpu.VMEM((_HBLK, _W, _C), jnp.float32),
            pltpu.VMEM((_HBLK, _W, _C), jnp.float32),
            pltpu.VMEM((_HBLK, _W, _C), jnp.float32),
            pltpu.VMEM((_HBLK, _W, _C), jnp.float32),
            pltpu.VMEM((_HBLK, _W, _C), jnp.float32),
            pltpu.VMEM((_HBLK, _W, _C), jnp.float32),
            pltpu.VMEM((_HBLK, _W, _C), jnp.float32),
            pltpu.VMEM((_B, _C), jnp.float32),
            pltpu.SemaphoreType.DMA((8,)),
        ],
    )(xt, w1, b1, w2, b2)
    return vals, idxs


# speedup vs baseline: 1.0154x; 1.0154x over previous
import jax
import jax.numpy as jnp
from jax.experimental import pallas as pl
from jax.experimental.pallas import tpu as pltpu

_B = 4
_C = 384
_H = 224
_W = 224
_HW = _H * _W
_R = 48
_E = 64
_K = 8
_EPS = 1e-5

_HBLK = 14
_CPB = _H // _HBLK
_NCHUNK = _B * _CPB
_NBUF = 8


def _route(pooled, w1_ref, b1_ref, w2_ref, b2_ref, vals_ref, idx_ref):
    hid = jax.lax.dot_general(pooled, w1_ref[...], (((1,), (1,)), ((), ())),
                              preferred_element_type=jnp.float32)
    hid = hid + b1_ref[...]
    hid = hid * jax.nn.sigmoid(hid)
    logits = jax.lax.dot_general(hid, w2_ref[...], (((1,), (1,)), ((), ())),
                                 preferred_element_type=jnp.float32)
    logits = logits + b2_ref[...]
    m = jnp.max(logits, axis=1, keepdims=True)
    e = jnp.exp(logits - m)
    probs = e / jnp.sum(e, axis=1, keepdims=True)

    pa = probs[:, :, None]
    pb = probs[:, None, :]
    ks = jax.lax.broadcasted_iota(jnp.int32, (_B, _E, _E), 1)
    js = jax.lax.broadcasted_iota(jnp.int32, (_B, _E, _E), 2)
    beats = (pa > pb) | ((pa == pb) & (ks < js))
    rank = jnp.sum(beats.astype(jnp.int32), axis=1)

    iota = jax.lax.broadcasted_iota(jnp.int32, (_B, _E), 1)
    vals = []
    idxs = []
    for s in range(_K):
        sel = rank == s
        vals.append(jnp.sum(jnp.where(sel, probs, 0.0), axis=1, keepdims=True))
        idxs.append(jnp.sum(jnp.where(sel, iota, 0), axis=1, keepdims=True))
    v = jnp.concatenate(vals, axis=1)
    i = jnp.concatenate(idxs, axis=1)
    ssum = jnp.sum(v, axis=1, keepdims=True) + 1e-6
    vals_ref[...] = v / ssum
    idx_ref[...] = i


def _body(xt_ref, w1_ref, b1_ref, w2_ref, b2_ref, vals_ref, idx_ref,
          r0, r1, r2, r3, r4, r5, r6, r7, sums_ref, sems):
    rings = (r0, r1, r2, r3, r4, r5, r6, r7)

    def start(i, j):
        b = i // _CPB
        h = i % _CPB
        pltpu.make_async_copy(
            xt_ref.at[b, pl.ds(h * _HBLK, _HBLK)],
            rings[j],
            sems.at[j],
        ).start()

    for i in range(_NBUF - 1):
        start(i, i)

    def group(g, _):
        for j in range(_NBUF):
            i = _NBUF * g + j
            pltpu.make_async_copy(
                xt_ref.at[0, pl.ds(0, _HBLK)],
                rings[j],
                sems.at[j],
            ).wait()

            @pl.when(i + _NBUF - 1 < _NCHUNK)
            def _prefetch():
                start(i + _NBUF - 1, (j + _NBUF - 1) % _NBUF)

            s = jnp.sum(rings[j][...], axis=0)
            part = jnp.sum(s, axis=0)
            b = i // _CPB

            @pl.when(i % _CPB == 0)
            def _init():
                sums_ref[b, :] = part

            @pl.when(i % _CPB != 0)
            def _acc():
                sums_ref[b, :] += part
        return _

    jax.lax.fori_loop(0, _NCHUNK // _NBUF, group, None)
    _route(sums_ref[...], w1_ref, b1_ref, w2_ref, b2_ref, vals_ref, idx_ref)


@jax.jit
def kernel(x, W1, gamma1, beta1, mean1, var1, W2, gamma2, beta2, mean2, var2):
    s1 = gamma1 * jax.lax.rsqrt(var1 + _EPS)
    s2 = gamma2 * jax.lax.rsqrt(var2 + _EPS)
    w1 = (W1 * s1[:, None]) * (1.0 / _HW)
    b1 = (beta1 - mean1 * s1)[None, :]
    w2 = W2 * s2[:, None]
    b2 = (beta2 - mean2 * s2)[None, :]

    xt = jnp.transpose(x, (0, 2, 3, 1))
    vals, idxs = pl.pallas_call(
        _body,
        in_specs=[
            pl.BlockSpec(memory_space=pl.ANY),
            pl.BlockSpec(memory_space=pltpu.VMEM),
            pl.BlockSpec(memory_space=pltpu.VMEM),
            pl.BlockSpec(memory_space=pltpu.VMEM),
            pl.BlockSpec(memory_space=pltpu.VMEM),
        ],
        out_specs=(
            pl.BlockSpec(memory_space=pltpu.VMEM),
            pl.BlockSpec(memory_space=pltpu.VMEM),
        ),
        out_shape=(
            jax.ShapeDtypeStruct((_B, _K), jnp.float32),
            jax.ShapeDtypeStruct((_B, _K), jnp.int32),
        ),
        scratch_shapes=[
            pltpu.VMEM((_HBLK, _W, _C), jnp.float32),
            pltpu.VMEM((_HBLK, _W, _C), jnp.float32),
            pltpu.VMEM((_HBLK, _W, _C), jnp.float32),
            pltpu.VMEM((_HBLK, _W, _C), jnp.float32),
            pltpu.VMEM((_HBLK, _W, _C), jnp.float32),
            pltpu.VMEM((_HBLK, _W, _C), jnp.float32),
            pltpu.VMEM((_HBLK, _W, _C), jnp.float32),
            pltpu.VMEM((_HBLK, _W, _C), jnp.float32),
            pltpu.VMEM((_B, _C), jnp.float32),
            pltpu.SemaphoreType.DMA((_NBUF,)),
        ],
    )(xt, w1, b1, w2, b2)
    return vals, idxs


# speedup vs baseline: 1.0221x; 1.0066x over previous
import jax
import jax.numpy as jnp
from jax.experimental import pallas as pl
from jax.experimental.pallas import tpu as pltpu

_B = 4
_C = 384
_H = 224
_W = 224
_HW = _H * _W
_R = 48
_E = 64
_K = 8
_EPS = 1e-5

_HBLK = 14
_CPB = _H // _HBLK
_NCHUNK = _B * _CPB
_NBUF = 4


def _route(pooled, w1_ref, b1_ref, w2_ref, b2_ref, vals_ref, idx_ref):
    hid = jax.lax.dot_general(pooled, w1_ref[...], (((1,), (1,)), ((), ())),
                              preferred_element_type=jnp.float32)
    hid = hid + b1_ref[...]
    hid = hid * jax.nn.sigmoid(hid)
    logits = jax.lax.dot_general(hid, w2_ref[...], (((1,), (1,)), ((), ())),
                                 preferred_element_type=jnp.float32)
    logits = logits + b2_ref[...]
    m = jnp.max(logits, axis=1, keepdims=True)
    e = jnp.exp(logits - m)
    probs = e / jnp.sum(e, axis=1, keepdims=True)

    pa = probs[:, :, None]
    pb = probs[:, None, :]
    ks = jax.lax.broadcasted_iota(jnp.int32, (_B, _E, _E), 1)
    js = jax.lax.broadcasted_iota(jnp.int32, (_B, _E, _E), 2)
    beats = (pa > pb) | ((pa == pb) & (ks < js))
    rank = jnp.sum(beats.astype(jnp.int32), axis=1)

    iota = jax.lax.broadcasted_iota(jnp.int32, (_B, _E), 1)
    vals = []
    idxs = []
    for s in range(_K):
        sel = rank == s
        vals.append(jnp.sum(jnp.where(sel, probs, 0.0), axis=1, keepdims=True))
        idxs.append(jnp.sum(jnp.where(sel, iota, 0), axis=1, keepdims=True))
    v = jnp.concatenate(vals, axis=1)
    i = jnp.concatenate(idxs, axis=1)
    ssum = jnp.sum(v, axis=1, keepdims=True) + 1e-6
    vals_ref[...] = v / ssum
    idx_ref[...] = i


def _body(xt_ref, w1_ref, b1_ref, w2_ref, b2_ref, vals_ref, idx_ref,
          r0, r1, r2, r3, sums_ref, sems):
    rings = (r0, r1, r2, r3)

    def start(i, j):
        b = i // _CPB
        h = i % _CPB
        pltpu.make_async_copy(
            xt_ref.at[b, pl.ds(h * _HBLK, _HBLK)],
            rings[j],
            sems.at[j],
        ).start()

    for i in range(_NBUF - 1):
        start(i, i)

    def group(g, _):
        for j in range(_NBUF):
            i = _NBUF * g + j
            pltpu.make_async_copy(
                xt_ref.at[0, pl.ds(0, _HBLK)],
                rings[j],
                sems.at[j],
            ).wait()

            @pl.when(i + _NBUF - 1 < _NCHUNK)
            def _prefetch():
                start(i + _NBUF - 1, (j + _NBUF - 1) % _NBUF)

            s = jnp.sum(rings[j][...], axis=0)
            part = jnp.sum(s, axis=0)
            b = i // _CPB

            @pl.when(i % _CPB == 0)
            def _init():
                sums_ref[b, :] = part

            @pl.when(i % _CPB != 0)
            def _acc():
                sums_ref[b, :] += part
        return _

    jax.lax.fori_loop(0, _NCHUNK // _NBUF, group, None)
    _route(sums_ref[...], w1_ref, b1_ref, w2_ref, b2_ref, vals_ref, idx_ref)


@jax.jit
def kernel(x, W1, gamma1, beta1, mean1, var1, W2, gamma2, beta2, mean2, var2):
    s1 = gamma1 * jax.lax.rsqrt(var1 + _EPS)
    s2 = gamma2 * jax.lax.rsqrt(var2 + _EPS)
    w1 = (W1 * s1[:, None]) * (1.0 / _HW)
    b1 = (beta1 - mean1 * s1)[None, :]
    w2 = W2 * s2[:, None]
    b2 = (beta2 - mean2 * s2)[None, :]

    xt = jnp.transpose(x, (0, 2, 3, 1))
    vals, idxs = pl.pallas_call(
        _body,
        in_specs=[
            pl.BlockSpec(memory_space=pl.ANY),
            pl.BlockSpec(memory_space=pltpu.VMEM),
            pl.BlockSpec(memory_space=pltpu.VMEM),
            pl.BlockSpec(memory_space=pltpu.VMEM),
            pl.BlockSpec(memory_space=pltpu.VMEM),
        ],
        out_specs=(
            pl.BlockSpec(memory_space=pltpu.VMEM),
            pl.BlockSpec(memory_space=pltpu.VMEM),
        ),
        out_shape=(
            jax.ShapeDtypeStruct((_B, _K), jnp.float32),
            jax.ShapeDtypeStruct((_B, _K), jnp.int32),
        ),
        scratch_shapes=[
            pltpu.VMEM((_HBLK, _W, _C), jnp.float32),
            pltpu.VMEM((_HBLK, _W, _C), jnp.float32),
            pltpu.VMEM((_HBLK, _W, _C), jnp.float32),
            pltpu.VMEM((_HBLK, _W, _C), jnp.float32),
            pltpu.VMEM((_B, _C), jnp.float32),
            pltpu.SemaphoreType.DMA((_NBUF,)),
        ],
    )(xt, w1, b1, w2, b2)
    return vals, idxs
